# baseline (speedup 1.0000x reference)
import jax
import jax.numpy as jnp
from jax import lax
from jax.experimental import pallas as pl
from jax.experimental.pallas import tpu as pltpu
from jax.experimental.pallas import tpu_sc as plsc

S, B, D = 8192, 4, 1024
ROWS = S * B
NC, NS = 2, 16
NW = NC * NS
POS_PER_W = S // NW
CP = 8
CR = CP * B
NCHUNK = POS_PER_W // CP
NV = D // 16


def _body(x_hbm, pos_hbm, out_hbm, xbuf, pbuf):
    wid = lax.axis_index("s") * NC + lax.axis_index("c")
    s_base = wid * POS_PER_W

    def chunk_body(j, carry):
        s0 = s_base + j * CP
        r0 = s0 * B
        pltpu.sync_copy(x_hbm.at[pl.ds(r0, CR)], xbuf)
        pltpu.sync_copy(pos_hbm.at[pl.ds(s0, CP)], pbuf)

        def vec_body(v, c2):
            off = v * 16
            sl = pl.ds(off, 16)
            for p in range(CP):
                pv = pbuf[p, sl]
                for b in range(B):
                    xbuf[p * B + b, sl] = xbuf[p * B + b, sl] + pv
            return c2

        lax.fori_loop(0, NV, vec_body, 0, unroll=2)
        pltpu.sync_copy(xbuf, out_hbm.at[pl.ds(r0, CR)])
        return carry

    lax.fori_loop(0, NCHUNK, chunk_body, 0)


@jax.jit
def _run(x2, pos_emb):
    mesh = plsc.VectorSubcoreMesh(core_axis_name="c", subcore_axis_name="s")
    return pl.kernel(
        _body,
        out_type=jax.ShapeDtypeStruct((ROWS, D), jnp.float32),
        mesh=mesh,
        scratch_types=[
            pltpu.VMEM((CR, D), jnp.float32),
            pltpu.VMEM((CP, D), jnp.float32),
        ],
    )(x2, pos_emb)


def kernel(x, pos_emb):
    out2 = _run(x.reshape(ROWS, D), pos_emb)
    return out2.reshape(S, B, D)


# speedup vs baseline: 1.7148x; 1.7148x over previous
import jax
import jax.numpy as jnp
from jax import lax
from jax.experimental import pallas as pl
from jax.experimental.pallas import tpu as pltpu
from jax.experimental.pallas import tpu_sc as plsc

S, B, D = 8192, 4, 1024
ROWS = S * B
NC, NS = 2, 16
NW = NC * NS
POS_PER_W = S // NW
CP = 8
CR = CP * B
NCHUNK = POS_PER_W // CP
NV = D // 16
NBUF = 3


def _body(x_hbm, pos_hbm, out_hbm, xbuf, pbuf, semx, semp, semo):
    wid = lax.axis_index("s") * NC + lax.axis_index("c")
    s_base = wid * POS_PER_W

    def issue_in(j, slot):
        s0 = s_base + j * CP
        pltpu.async_copy(x_hbm.at[pl.ds(s0 * B, CR)], xbuf.at[slot],
                         semx.at[slot])
        pltpu.async_copy(pos_hbm.at[pl.ds(s0, CP)], pbuf.at[slot],
                         semp.at[slot])

    def wait_in(slot):
        pltpu.make_async_copy(x_hbm.at[pl.ds(0, CR)], xbuf.at[slot],
                              semx.at[slot]).wait()
        pltpu.make_async_copy(pos_hbm.at[pl.ds(0, CP)], pbuf.at[slot],
                              semp.at[slot]).wait()

    def issue_out(j, slot):
        s0 = s_base + j * CP
        pltpu.async_copy(xbuf.at[slot], out_hbm.at[pl.ds(s0 * B, CR)],
                         semo.at[slot])

    def wait_out(slot):
        pltpu.make_async_copy(xbuf.at[slot], out_hbm.at[pl.ds(0, CR)],
                              semo.at[slot]).wait()

    issue_in(0, 0)
    issue_in(1, 1)

    def chunk_body(j, carry):
        slot = lax.rem(j, NBUF)
        nslot = lax.rem(j + 2, NBUF)
        wait_in(slot)

        def vec_body(v, c2):
            sl = pl.ds(v * 16, 16)
            for p in range(CP):
                pv = pbuf[slot, p, sl]
                for b in range(B):
                    r = p * B + b
                    xbuf[slot, r, sl] = xbuf[slot, r, sl] + pv
            return c2

        lax.fori_loop(0, NV, vec_body, 0, unroll=2)
        issue_out(j, slot)

        @pl.when(jnp.logical_and(j >= 1, j + 2 < NCHUNK))
        def _():
            wait_out(nslot)

        @pl.when(j + 2 < NCHUNK)
        def _():
            issue_in(j + 2, nslot)

        return carry

    lax.fori_loop(0, NCHUNK, chunk_body, 0)
    for s in range(NBUF):
        wait_out(s)


@jax.jit
def _run(x2, pos_emb):
    mesh = plsc.VectorSubcoreMesh(core_axis_name="c", subcore_axis_name="s")
    return pl.kernel(
        _body,
        out_type=jax.ShapeDtypeStruct((ROWS, D), jnp.float32),
        mesh=mesh,
        scratch_types=[
            pltpu.VMEM((NBUF, CR, D), jnp.float32),
            pltpu.VMEM((NBUF, CP, D), jnp.float32),
            pltpu.SemaphoreType.DMA((NBUF,)),
            pltpu.SemaphoreType.DMA((NBUF,)),
            pltpu.SemaphoreType.DMA((NBUF,)),
        ],
    )(x2, pos_emb)


def kernel(x, pos_emb):
    out2 = _run(x.reshape(ROWS, D), pos_emb)
    return out2.reshape(S, B, D)


# speedup vs baseline: 5.2001x; 3.0324x over previous
import jax
import jax.numpy as jnp
from jax import lax
from jax.experimental import pallas as pl
from jax.experimental.pallas import tpu as pltpu
from jax.experimental.pallas import tpu_sc as plsc

S, B, D = 8192, 4, 1024
NC, NS = 2, 16
NW = NC * NS
POS_PER_W = S // NW
CP = 8
NCHUNK = POS_PER_W // CP
NV = D // 16
NBUF = 3


def _body(x_hbm, pos_hbm, out_hbm, xbuf, pbuf, semx, semp, semo):
    wid = lax.axis_index("s") * NC + lax.axis_index("c")
    s_base = wid * POS_PER_W

    def issue_in(j, slot):
        s0 = s_base + j * CP
        pltpu.async_copy(x_hbm.at[pl.ds(s0, CP)], xbuf.at[slot],
                         semx.at[slot])
        pltpu.async_copy(pos_hbm.at[pl.ds(s0, CP)], pbuf.at[slot],
                         semp.at[slot])

    def wait_in(slot):
        pltpu.make_async_copy(x_hbm.at[pl.ds(0, CP)], xbuf.at[slot],
                              semx.at[slot]).wait()
        pltpu.make_async_copy(pos_hbm.at[pl.ds(0, CP)], pbuf.at[slot],
                              semp.at[slot]).wait()

    def issue_out(j, slot):
        s0 = s_base + j * CP
        pltpu.async_copy(xbuf.at[slot], out_hbm.at[pl.ds(s0, CP)],
                         semo.at[slot])

    def wait_out(slot):
        pltpu.make_async_copy(xbuf.at[slot], out_hbm.at[pl.ds(0, CP)],
                              semo.at[slot]).wait()

    issue_in(0, 0)
    issue_in(1, 1)

    def chunk_body(j, carry):
        slot = lax.rem(j, NBUF)
        nslot = lax.rem(j + 2, NBUF)
        wait_in(slot)

        def vec_body(v, c2):
            sl = pl.ds(v * 16, 16)
            for p in range(CP):
                pv = pbuf[slot, p, sl]
                for b in range(B):
                    xbuf[slot, p, b, sl] = xbuf[slot, p, b, sl] + pv
            return c2

        lax.fori_loop(0, NV, vec_body, 0, unroll=2)
        issue_out(j, slot)

        @pl.when(jnp.logical_and(j >= 1, j + 2 < NCHUNK))
        def _():
            wait_out(nslot)

        @pl.when(j + 2 < NCHUNK)
        def _():
            issue_in(j + 2, nslot)

        return carry

    lax.fori_loop(0, NCHUNK, chunk_body, 0)
    for s in range(NBUF):
        wait_out(s)


@jax.jit
def kernel(x, pos_emb):
    mesh = plsc.VectorSubcoreMesh(core_axis_name="c", subcore_axis_name="s")
    return pl.kernel(
        _body,
        out_type=jax.ShapeDtypeStruct((S, B, D), jnp.float32),
        mesh=mesh,
        scratch_types=[
            pltpu.VMEM((NBUF, CP, B, D), jnp.float32),
            pltpu.VMEM((NBUF, CP, D), jnp.float32),
            pltpu.SemaphoreType.DMA((NBUF,)),
            pltpu.SemaphoreType.DMA((NBUF,)),
            pltpu.SemaphoreType.DMA((NBUF,)),
        ],
    )(x, pos_emb)


# speedup vs baseline: 5.6475x; 1.0860x over previous
import jax
import jax.numpy as jnp
from jax import lax
from jax.experimental import pallas as pl
from jax.experimental.pallas import tpu as pltpu
from jax.experimental.pallas import tpu_sc as plsc

S, B, D = 8192, 4, 1024
NC, NS = 2, 16
NW = NC * NS
POS_PER_W = S // NW
CP = 8
NCHUNK = POS_PER_W // CP
NV = D // 16
NBUF = 3


def _body(x_hbm, pos_hbm, out_hbm, xbuf, pbuf, semx, semp, semo):
    wid = lax.axis_index("s") * NC + lax.axis_index("c")
    s_base = wid * POS_PER_W

    def issue_in(j, slot):
        s0 = s_base + j * CP
        pltpu.async_copy(x_hbm.at[pl.ds(s0, CP)], xbuf.at[slot],
                         semx.at[slot])
        pltpu.async_copy(pos_hbm.at[pl.ds(s0, CP)], pbuf.at[slot],
                         semp.at[slot])

    def wait_in(slot):
        pltpu.make_async_copy(x_hbm.at[pl.ds(0, CP)], xbuf.at[slot],
                              semx.at[slot]).wait()
        pltpu.make_async_copy(pos_hbm.at[pl.ds(0, CP)], pbuf.at[slot],
                              semp.at[slot]).wait()

    def issue_out(j, slot):
        s0 = s_base + j * CP
        pltpu.async_copy(xbuf.at[slot], out_hbm.at[pl.ds(s0, CP)],
                         semo.at[slot])

    def wait_out(slot):
        pltpu.make_async_copy(xbuf.at[slot], out_hbm.at[pl.ds(0, CP)],
                              semo.at[slot]).wait()

    issue_in(0, 0)
    issue_in(1, 1)

    def chunk_body(j, carry):
        slot = lax.rem(j, NBUF)
        nslot = lax.rem(j + 2, NBUF)
        wait_in(slot)
        xs = xbuf.at[slot]
        ps = pbuf.at[slot]

        @plsc.parallel_loop(0, NV, unroll=4)
        def vec_body(v):
            sl = pl.ds(v * 16, 16)
            for p in range(CP):
                pv = ps[p, sl]
                for b in range(B):
                    xs[p, b, sl] = xs[p, b, sl] + pv
        issue_out(j, slot)

        @pl.when(jnp.logical_and(j >= 1, j + 2 < NCHUNK))
        def _():
            wait_out(nslot)

        @pl.when(j + 2 < NCHUNK)
        def _():
            issue_in(j + 2, nslot)

        return carry

    lax.fori_loop(0, NCHUNK, chunk_body, 0)
    for s in range(NBUF):
        wait_out(s)


@jax.jit
def kernel(x, pos_emb):
    mesh = plsc.VectorSubcoreMesh(core_axis_name="c", subcore_axis_name="s")
    return pl.kernel(
        _body,
        out_type=jax.ShapeDtypeStruct((S, B, D), jnp.float32),
        mesh=mesh,
        scratch_types=[
            pltpu.VMEM((NBUF, CP, B, D), jnp.float32),
            pltpu.VMEM((NBUF, CP, D), jnp.float32),
            pltpu.SemaphoreType.DMA((NBUF,)),
            pltpu.SemaphoreType.DMA((NBUF,)),
            pltpu.SemaphoreType.DMA((NBUF,)),
        ],
    )(x, pos_emb)


# speedup vs baseline: 5.8284x; 1.0320x over previous
import jax
import jax.numpy as jnp
from jax import lax
from jax.experimental import pallas as pl
from jax.experimental.pallas import tpu as pltpu
from jax.experimental.pallas import tpu_sc as plsc

S, B, D = 8192, 4, 1024
NC, NS = 2, 16
NW = NC * NS
POS_PER_W = S // NW
CP = 4
NCHUNK = POS_PER_W // CP
NV = D // 16
NBUF = 6
LOOK = NBUF - 1


def _body(x_hbm, pos_hbm, out_hbm, xbuf, pbuf, semx, semp, semo):
    wid = lax.axis_index("s") * NC + lax.axis_index("c")
    s_base = wid * POS_PER_W

    def issue_in(j, slot):
        s0 = s_base + j * CP
        pltpu.async_copy(x_hbm.at[pl.ds(s0, CP)], xbuf.at[slot],
                         semx.at[slot])
        pltpu.async_copy(pos_hbm.at[pl.ds(s0, CP)], pbuf.at[slot],
                         semp.at[slot])

    def wait_in(slot):
        pltpu.make_async_copy(x_hbm.at[pl.ds(0, CP)], xbuf.at[slot],
                              semx.at[slot]).wait()
        pltpu.make_async_copy(pos_hbm.at[pl.ds(0, CP)], pbuf.at[slot],
                              semp.at[slot]).wait()

    def issue_out(j, slot):
        s0 = s_base + j * CP
        pltpu.async_copy(xbuf.at[slot], out_hbm.at[pl.ds(s0, CP)],
                         semo.at[slot])

    def wait_out(slot):
        pltpu.make_async_copy(xbuf.at[slot], out_hbm.at[pl.ds(0, CP)],
                              semo.at[slot]).wait()

    for k in range(LOOK):
        issue_in(k, k)

    def chunk_body(j, carry):
        slot = lax.rem(j, NBUF)
        nslot = lax.rem(j + LOOK, NBUF)
        wait_in(slot)
        xs = xbuf.at[slot]
        ps = pbuf.at[slot]

        @plsc.parallel_loop(0, NV, unroll=4)
        def vec_body(v):
            sl = pl.ds(v * 16, 16)
            for p in range(CP):
                pv = ps[p, sl]
                for b in range(B):
                    xs[p, b, sl] = xs[p, b, sl] + pv
        issue_out(j, slot)

        @pl.when(jnp.logical_and(j >= 1, j + LOOK < NCHUNK))
        def _():
            wait_out(nslot)

        @pl.when(j + LOOK < NCHUNK)
        def _():
            issue_in(j + LOOK, nslot)

        return carry

    lax.fori_loop(0, NCHUNK, chunk_body, 0)
    for s in range(NBUF):
        wait_out(s)


@jax.jit
def kernel(x, pos_emb):
    mesh = plsc.VectorSubcoreMesh(core_axis_name="c", subcore_axis_name="s")
    return pl.kernel(
        _body,
        out_type=jax.ShapeDtypeStruct((S, B, D), jnp.float32),
        mesh=mesh,
        scratch_types=[
            pltpu.VMEM((NBUF, CP, B, D), jnp.float32),
            pltpu.VMEM((NBUF, CP, D), jnp.float32),
            pltpu.SemaphoreType.DMA((NBUF,)),
            pltpu.SemaphoreType.DMA((NBUF,)),
            pltpu.SemaphoreType.DMA((NBUF,)),
        ],
    )(x, pos_emb)
